# baseline (speedup 1.0000x reference)
import functools

import jax
import jax.numpy as jnp
from jax import lax
from jax.experimental import pallas as pl
from jax.experimental.pallas import tpu as pltpu
from jax.experimental.pallas import tpu_sc as plsc

_NUM_CLASS = 100000
_CTX_DIM = 512
_N_CLS_CTX = 4
_PREFIX_LEN = 5
_SUFFIX_LEN = 68
_BATCH = 1024
_ROW = _N_CLS_CTX * _CTX_DIM
_SEQ = _PREFIX_LEN + _N_CLS_CTX + _SUFFIX_LEN

_BB = 128


@functools.lru_cache(maxsize=1)
def _make_sc_gather():
    info = plsc.get_sparse_core_info()
    nc, ns = info.num_cores, info.num_subcores
    nw = nc * ns
    b_per_w = _BATCH // nw
    mesh = plsc.VectorSubcoreMesh(core_axis_name="c", subcore_axis_name="s")

    @functools.partial(
        pl.kernel,
        mesh=mesh,
        out_type=jax.ShapeDtypeStruct((_BATCH, _ROW), jnp.float32),
        scratch_types=[
            pltpu.VMEM((b_per_w,), jnp.int32),
            pltpu.VMEM((b_per_w, _ROW), jnp.float32),
            pltpu.SemaphoreType.DMA,
        ],
    )
    def sc_gather(idx_hbm, table_hbm, out_hbm, idx_v, rows_v, sem):
        wid = lax.axis_index("s") * nc + lax.axis_index("c")
        base = wid * b_per_w
        pltpu.sync_copy(idx_hbm.at[pl.ds(base, b_per_w)], idx_v)
        pltpu.async_copy(table_hbm.at[idx_v], rows_v, sem).wait()
        pltpu.sync_copy(rows_v, out_hbm.at[pl.ds(base, b_per_w)])

    return sc_gather


def _assemble_body(p_ref, s_ref, g_ref, o_ref):
    o_ref[:, 0:_PREFIX_LEN, :] = jnp.broadcast_to(
        p_ref[...], (_BB, _PREFIX_LEN, _CTX_DIM))
    o_ref[:, _PREFIX_LEN:_PREFIX_LEN + _N_CLS_CTX, :] = g_ref[...]
    o_ref[:, _PREFIX_LEN + _N_CLS_CTX:, :] = jnp.broadcast_to(
        s_ref[...], (_BB, _SUFFIX_LEN, _CTX_DIM))


_assemble = pl.pallas_call(
    _assemble_body,
    grid=(_BATCH // _BB,),
    in_specs=[
        pl.BlockSpec((1, _PREFIX_LEN, _CTX_DIM), lambda i: (0, 0, 0)),
        pl.BlockSpec((1, _SUFFIX_LEN, _CTX_DIM), lambda i: (0, 0, 0)),
        pl.BlockSpec((_BB, _N_CLS_CTX, _CTX_DIM), lambda i: (i, 0, 0)),
    ],
    out_specs=pl.BlockSpec((_BB, _SEQ, _CTX_DIM), lambda i: (i, 0, 0)),
    out_shape=jax.ShapeDtypeStruct((_BATCH, _SEQ, _CTX_DIM), jnp.float32),
)


def kernel(get_train, label, cls_ctx, token_prefix, token_suffix):
    table = cls_ctx.reshape(_NUM_CLASS, _ROW)
    gathered = _make_sc_gather()(label, table)
    gathered = gathered.reshape(_BATCH, _N_CLS_CTX, _CTX_DIM)
    return _assemble(token_prefix, token_suffix, gathered)


# speedup vs baseline: 7.9595x; 7.9595x over previous
import functools

import jax
import jax.numpy as jnp
from jax import lax
from jax.experimental import pallas as pl
from jax.experimental.pallas import tpu as pltpu
from jax.experimental.pallas import tpu_sc as plsc

_NUM_CLASS = 100000
_CTX_DIM = 512
_N_CLS_CTX = 4
_PREFIX_LEN = 5
_SUFFIX_LEN = 68
_BATCH = 1024
_SEQ = _PREFIX_LEN + _N_CLS_CTX + _SUFFIX_LEN
_CTX_START = _PREFIX_LEN
_CTX_END = _PREFIX_LEN + _N_CLS_CTX


@functools.lru_cache(maxsize=1)
def _make_sc_gather():
    info = plsc.get_sparse_core_info()
    nc, ns = info.num_cores, info.num_subcores
    nw = nc * ns
    b_per_w = _BATCH // nw
    mesh = plsc.VectorSubcoreMesh(core_axis_name="c", subcore_axis_name="s")

    @functools.partial(
        pl.kernel,
        mesh=mesh,
        out_type=jax.ShapeDtypeStruct((_N_CLS_CTX, _BATCH, _CTX_DIM),
                                      jnp.float32),
        scratch_types=[
            pltpu.VMEM((b_per_w,), jnp.int32),
            pltpu.VMEM((b_per_w, _N_CLS_CTX, _CTX_DIM), jnp.float32),
            pltpu.SemaphoreType.DMA,
        ],
    )
    def sc_gather(idx_hbm, table_hbm, out_hbm, idx_v, rows_v, sem):
        wid = lax.axis_index("s") * nc + lax.axis_index("c")
        base = wid * b_per_w
        pltpu.sync_copy(idx_hbm.at[pl.ds(base, b_per_w)], idx_v)
        pltpu.async_copy(table_hbm.at[idx_v], rows_v, sem).wait()
        for c in range(_N_CLS_CTX):
            pltpu.sync_copy(rows_v.at[:, c, :],
                            out_hbm.at[c, pl.ds(base, b_per_w), :])

    return sc_gather


def _assemble_body(p_ref, s_ref, g_ref, o_ref):
    j = pl.program_id(0)

    @pl.when(j < _CTX_START)
    def _prefix():
        o_ref[...] = jnp.broadcast_to(p_ref[...], (1, _BATCH, _CTX_DIM))

    @pl.when(jnp.logical_and(j >= _CTX_START, j < _CTX_END))
    def _gathered():
        o_ref[...] = g_ref[...]

    @pl.when(j >= _CTX_END)
    def _suffix():
        o_ref[...] = jnp.broadcast_to(s_ref[...], (1, _BATCH, _CTX_DIM))


_assemble = pl.pallas_call(
    _assemble_body,
    grid=(_SEQ,),
    in_specs=[
        pl.BlockSpec((1, 1, _CTX_DIM),
                     lambda j: (jnp.clip(j, 0, _PREFIX_LEN - 1), 0, 0)),
        pl.BlockSpec((1, 1, _CTX_DIM),
                     lambda j: (jnp.clip(j - _CTX_END, 0, _SUFFIX_LEN - 1), 0,
                                0)),
        pl.BlockSpec((1, _BATCH, _CTX_DIM),
                     lambda j: (jnp.clip(j - _CTX_START, 0, _N_CLS_CTX - 1),
                                0, 0)),
    ],
    out_specs=pl.BlockSpec((1, _BATCH, _CTX_DIM), lambda j: (j, 0, 0)),
    out_shape=jax.ShapeDtypeStruct((_SEQ, _BATCH, _CTX_DIM), jnp.float32),
)


def kernel(get_train, label, cls_ctx, token_prefix, token_suffix):
    gathered = _make_sc_gather()(label, cls_ctx)
    prefix = token_prefix.reshape(_PREFIX_LEN, 1, _CTX_DIM)
    suffix = token_suffix.reshape(_SUFFIX_LEN, 1, _CTX_DIM)
    out_seq_major = _assemble(prefix, suffix, gathered)
    return out_seq_major.transpose(1, 0, 2)


# speedup vs baseline: 8.3413x; 1.0480x over previous
import functools

import jax
import jax.numpy as jnp
from jax import lax
from jax.experimental import pallas as pl
from jax.experimental.pallas import tpu as pltpu
from jax.experimental.pallas import tpu_sc as plsc

_NUM_CLASS = 100000
_CTX_DIM = 512
_N_CLS_CTX = 4
_PREFIX_LEN = 5
_SUFFIX_LEN = 68
_BATCH = 1024
_SEQ = _PREFIX_LEN + _N_CLS_CTX + _SUFFIX_LEN
_CTX_START = _PREFIX_LEN
_CTX_END = _PREFIX_LEN + _N_CLS_CTX


@functools.lru_cache(maxsize=1)
def _make_sc_gather():
    info = plsc.get_sparse_core_info()
    nc, ns = info.num_cores, info.num_subcores
    nw = nc * ns
    b_per_w = _BATCH // nw
    mesh = plsc.VectorSubcoreMesh(core_axis_name="c", subcore_axis_name="s")

    @functools.partial(
        pl.kernel,
        mesh=mesh,
        out_type=jax.ShapeDtypeStruct((_N_CLS_CTX, _BATCH, _CTX_DIM),
                                      jnp.float32),
        scratch_types=[
            pltpu.VMEM((b_per_w,), jnp.int32),
            pltpu.VMEM((b_per_w, _N_CLS_CTX, _CTX_DIM), jnp.float32),
            pltpu.SemaphoreType.DMA,
        ],
    )
    def sc_gather(idx_hbm, table_hbm, out_hbm, idx_v, rows_v, sem):
        wid = lax.axis_index("s") * nc + lax.axis_index("c")
        base = wid * b_per_w
        pltpu.sync_copy(idx_hbm.at[pl.ds(base, b_per_w)], idx_v)
        pltpu.async_copy(table_hbm.at[idx_v], rows_v, sem).wait()
        for c in range(_N_CLS_CTX):
            pltpu.sync_copy(rows_v.at[:, c, :],
                            out_hbm.at[c, pl.ds(base, b_per_w), :])

    return sc_gather


_TMPL_LEN = _PREFIX_LEN + _SUFFIX_LEN


def _assemble_body(t_ref, g_ref, o_ref):
    j = pl.program_id(0)
    mid = jnp.logical_and(j >= _CTX_START, j < _CTX_END)

    @pl.when(mid)
    def _gathered():
        o_ref[...] = g_ref[...]

    @pl.when(jnp.logical_not(mid))
    def _template():
        r = jnp.where(j < _CTX_START, j, j - _N_CLS_CTX)
        row = t_ref[pl.ds(r, 1), :]
        o_ref[0] = jnp.broadcast_to(row, (_BATCH, _CTX_DIM))


_assemble = pl.pallas_call(
    _assemble_body,
    grid=(_SEQ,),
    in_specs=[
        pl.BlockSpec((_TMPL_LEN, _CTX_DIM), lambda j: (0, 0)),
        pl.BlockSpec((1, _BATCH, _CTX_DIM),
                     lambda j: (jnp.clip(j - _CTX_START, 0, _N_CLS_CTX - 1),
                                0, 0)),
    ],
    out_specs=pl.BlockSpec((1, _BATCH, _CTX_DIM), lambda j: (j, 0, 0)),
    out_shape=jax.ShapeDtypeStruct((_SEQ, _BATCH, _CTX_DIM), jnp.float32),
)


def kernel(get_train, label, cls_ctx, token_prefix, token_suffix):
    gathered = _make_sc_gather()(label, cls_ctx)
    template = jnp.concatenate([token_prefix[0], token_suffix[0]], axis=0)
    out_seq_major = _assemble(template, gathered)
    return out_seq_major.transpose(1, 0, 2)
